# baseline (speedup 1.0000x reference)
import functools

import jax
import jax.numpy as jnp
from jax import lax
from jax.experimental import pallas as pl
from jax.experimental.pallas import tpu as pltpu
from jax.experimental.pallas import tpu_sc as plsc

_N = 10000
_NNZ = 320000
_NUM_GRAPHS = 64
_NEG = 0.01

_NT = 32
_PER_TILE = 10240
_PAD_NNZ = _PER_TILE * _NT
_RPS = 624
_REM0 = 16 * _RPS
_REM = _N - _REM0


def _make_spmm(width, ch, nbuf):
  nchunk = _PER_TILE // ch
  ngroup = nchunk // nbuf
  mesh = plsc.VectorSubcoreMesh(core_axis_name="c", subcore_axis_name="s")

  @functools.partial(
      pl.kernel,
      out_type=jax.ShapeDtypeStruct((2, _N, width), jnp.float32),
      mesh=mesh,
      scratch_types=[
          pltpu.VMEM((nbuf, ch), jnp.int32),
          pltpu.VMEM((nbuf, ch), jnp.int32),
          pltpu.VMEM((nbuf, ch), jnp.float32),
          [pltpu.VMEM((ch, width), jnp.float32)] * nbuf,
          pltpu.VMEM_SHARED((_N, width), jnp.float32),
          pltpu.SemaphoreType.DMA,
      ],
      compiler_params=pltpu.CompilerParams(use_tc_tiling_on_sc=False),
      name=f"spmm_sc_w{width}",
  )
  def spmm(y, colr, rowr, valr, zeros, out, colv, rowv, valv, gbufs, acc,
           gsem):
    c = lax.axis_index("c")
    s = lax.axis_index("s")
    wid = s * 2 + c
    r0 = s * _RPS
    pltpu.sync_copy(zeros.at[pl.ds(r0, _RPS)], acc.at[pl.ds(r0, _RPS)])

    @pl.when(s == 15)
    def _():
      pltpu.sync_copy(zeros.at[pl.ds(_REM0, _REM)], acc.at[pl.ds(_REM0, _REM)])
    plsc.subcore_barrier()

    def scale(gbuf, b):
      def scale16(t, carry2):
        vv = valv[b, pl.ds(t * 16, 16)]
        for e in range(16):
          v = vv[e]
          r = t * 16 + e
          for k in range(width // 16):
            sl = pl.ds(k * 16, 16)
            gbuf[r, sl] = gbuf[r, sl] * v
        return carry2

      lax.fori_loop(0, ch // 16, scale16, 0)

    def group(g, carry):
      j0 = g * nbuf
      pltpu.sync_copy(colr.at[wid, pl.ds(j0, nbuf)], colv)
      pltpu.sync_copy(rowr.at[wid, pl.ds(j0, nbuf)], rowv)
      pltpu.sync_copy(valr.at[wid, pl.ds(j0, nbuf)], valv)
      for b in range(nbuf):
        pltpu.async_copy(y.at[colv.at[b]], gbufs[b], gsem).wait()
      for b in range(nbuf):
        scale(gbufs[b], b)
      for b in range(nbuf):
        pltpu.sync_copy(gbufs[b], acc.at[rowv.at[b]], add=True)
      return carry

    lax.fori_loop(0, ngroup, group, 0)
    plsc.subcore_barrier()
    pltpu.sync_copy(acc.at[pl.ds(r0, _RPS)], out.at[c, pl.ds(r0, _RPS)])

    @pl.when(s == 15)
    def _():
      pltpu.sync_copy(acc.at[pl.ds(_REM0, _REM)],
                      out.at[c, pl.ds(_REM0, _REM)])

  return spmm, nchunk, ch


_spmm128, _NCH128, _CH128 = _make_spmm(128, 64, 2)
_spmm32, _NCH32, _CH32 = _make_spmm(32, 128, 4)

_BLK = 2000


def _mm_relu(p, w, wi, wo):
  def body(p_ref, w_ref, o_ref):
    s = p_ref[0] + p_ref[1]
    y = jnp.dot(s, w_ref[...], preferred_element_type=jnp.float32)
    o_ref[...] = jnp.where(y >= 0, y, y * _NEG)

  return pl.pallas_call(
      body,
      grid=(_N // _BLK,),
      in_specs=[
          pl.BlockSpec((2, _BLK, wi), lambda i: (0, i, 0)),
          pl.BlockSpec((wi, wo), lambda i: (0, 0)),
      ],
      out_specs=pl.BlockSpec((_BLK, wo), lambda i: (i, 0)),
      out_shape=jax.ShapeDtypeStruct((_N, wo), jnp.float32),
      name="mm_relu",
  )(p, w)


def _pool_softmax(p, w4p, batch2d):
  def body(p_ref, w_ref, b_ref, o_ref):
    s = p_ref[0] + p_ref[1]
    y = jnp.dot(s, w_ref[...], preferred_element_type=jnp.float32)
    h = jnp.where(y >= 0, y, y * _NEG)
    ccol = lax.broadcasted_iota(jnp.int32, (_N, 16), 1)
    h = jnp.where(ccol == 10, 1.0, h)
    gids = lax.broadcasted_iota(jnp.int32, (_NUM_GRAPHS, _N), 0)
    onehot = (gids == b_ref[...]).astype(jnp.float32)
    pooled = jnp.dot(onehot, h, preferred_element_type=jnp.float32,
                     precision=lax.Precision.HIGHEST)
    counts = jnp.maximum(pooled[:, 10:11], 1.0)
    means = pooled / counts
    gcol = lax.broadcasted_iota(jnp.int32, (_NUM_GRAPHS, 16), 1)
    valid = gcol < 10
    z = jnp.where(valid, means, -1e30)
    z = z - jnp.max(z, axis=1, keepdims=True)
    ez = jnp.where(valid, jnp.exp(z), 0.0)
    o_ref[...] = ez / jnp.sum(ez, axis=1, keepdims=True)

  return pl.pallas_call(
      body,
      in_specs=[
          pl.BlockSpec((2, _N, 32), lambda: (0, 0, 0)),
          pl.BlockSpec((32, 16), lambda: (0, 0)),
          pl.BlockSpec((1, _N), lambda: (0, 0)),
      ],
      out_specs=pl.BlockSpec((_NUM_GRAPHS, 16), lambda: (0, 0)),
      out_shape=jax.ShapeDtypeStruct((_NUM_GRAPHS, 16), jnp.float32),
      name="pool_softmax",
  )(p, w4p, batch2d)


def kernel(X1, L1_indices, L1_values, batch, W1, W2, W3, W4):
  row = L1_indices[0]
  col = L1_indices[1]
  npad = _PAD_NNZ - _NNZ
  pad_idx = (jnp.arange(npad, dtype=jnp.int32) * 131) % _N
  colf = jnp.concatenate([col, pad_idx])
  rowf = jnp.concatenate([row, pad_idx])
  valf = jnp.concatenate([L1_values, jnp.zeros((npad,), jnp.float32)])
  col128 = colf.reshape(_NT, _NCH128, _CH128)
  row128 = rowf.reshape(_NT, _NCH128, _CH128)
  val128 = valf.reshape(_NT, _NCH128, _CH128)
  col32 = colf.reshape(_NT, _NCH32, _CH32)
  row32 = rowf.reshape(_NT, _NCH32, _CH32)
  val32 = valf.reshape(_NT, _NCH32, _CH32)
  zeros128 = jnp.zeros((_N, 128), jnp.float32)
  zeros32 = jnp.zeros((_N, 32), jnp.float32)
  w4p = jnp.zeros((32, 16), jnp.float32).at[:, :10].set(W4)

  p = _spmm128(X1, col128, row128, val128, zeros128)
  h = _mm_relu(p, W1, 128, 32)
  p = _spmm32(h, col32, row32, val32, zeros32)
  h = _mm_relu(p, W2, 32, 32)
  p = _spmm32(h, col32, row32, val32, zeros32)
  h = _mm_relu(p, W3, 32, 32)
  p = _spmm32(h, col32, row32, val32, zeros32)
  out = _pool_softmax(p, w4p, batch.reshape(1, _N))
  return out[:, :10]


# speedup vs baseline: 1.1327x; 1.1327x over previous
import functools

import jax
import jax.numpy as jnp
from jax import lax
from jax.experimental import pallas as pl
from jax.experimental.pallas import tpu as pltpu
from jax.experimental.pallas import tpu_sc as plsc

_N = 10000
_NNZ = 320000
_NUM_GRAPHS = 64
_NEG = 0.01

_NT = 32
_PER_TILE = 10240
_PAD_NNZ = _PER_TILE * _NT
_RPS = 624
_REM0 = 16 * _RPS
_REM = _N - _REM0


def _make_spmm(width, ch, nbuf):
  nchunk = _PER_TILE // ch
  ngroup = nchunk // nbuf
  mesh = plsc.VectorSubcoreMesh(core_axis_name="c", subcore_axis_name="s")

  @functools.partial(
      pl.kernel,
      out_type=jax.ShapeDtypeStruct((2, _N, width), jnp.float32),
      mesh=mesh,
      scratch_types=[
          pltpu.VMEM((nbuf, ch), jnp.int32),
          pltpu.VMEM((nbuf, ch), jnp.int32),
          pltpu.VMEM((nbuf, ch), jnp.float32),
          [pltpu.VMEM((ch, width), jnp.float32)] * nbuf,
          pltpu.VMEM_SHARED((_N, width), jnp.float32),
          pltpu.SemaphoreType.DMA,
      ],
      compiler_params=pltpu.CompilerParams(use_tc_tiling_on_sc=False),
      name=f"spmm_sc_w{width}",
  )
  def spmm(y, colr, rowr, valr, zeros, out, colv, rowv, valv, gbufs, acc,
           gsem):
    c = lax.axis_index("c")
    s = lax.axis_index("s")
    wid = s * 2 + c
    r0 = s * _RPS
    pltpu.sync_copy(zeros.at[pl.ds(r0, _RPS)], acc.at[pl.ds(r0, _RPS)])

    @pl.when(s == 15)
    def _():
      pltpu.sync_copy(zeros.at[pl.ds(_REM0, _REM)], acc.at[pl.ds(_REM0, _REM)])
    plsc.subcore_barrier()

    def scale(gbuf, b):
      def scale16(t, carry2):
        vv = valv[b, pl.ds(t * 16, 16)]
        for e in range(16):
          v = vv[e]
          r = t * 16 + e
          for k in range(width // 16):
            sl = pl.ds(k * 16, 16)
            gbuf[r, sl] = gbuf[r, sl] * v
        return carry2

      lax.fori_loop(0, ch // 16, scale16, 0)

    def group(g, carry):
      j0 = g * nbuf
      pltpu.sync_copy(colr.at[wid, pl.ds(j0, nbuf)], colv)
      pltpu.sync_copy(rowr.at[wid, pl.ds(j0, nbuf)], rowv)
      pltpu.sync_copy(valr.at[wid, pl.ds(j0, nbuf)], valv)
      for b in range(nbuf):
        pltpu.async_copy(y.at[colv.at[b]], gbufs[b], gsem).wait()
      for b in range(nbuf):
        scale(gbufs[b], b)
      for b in range(nbuf):
        pltpu.sync_copy(gbufs[b], acc.at[rowv.at[b]], add=True)
      return carry

    lax.fori_loop(0, ngroup, group, 0)
    plsc.subcore_barrier()
    pltpu.sync_copy(acc.at[pl.ds(r0, _RPS)], out.at[c, pl.ds(r0, _RPS)])

    @pl.when(s == 15)
    def _():
      pltpu.sync_copy(acc.at[pl.ds(_REM0, _REM)],
                      out.at[c, pl.ds(_REM0, _REM)])

  return spmm, nchunk, ch


_spmm128, _NCH128, _CH128 = _make_spmm(128, 128, 2)
_spmm32, _NCH32, _CH32 = _make_spmm(32, 128, 4)

_BLK = 2000


def _mm_relu(p, w, wi, wo):
  def body(p_ref, w_ref, o_ref):
    s = p_ref[0] + p_ref[1]
    y = jnp.dot(s, w_ref[...], preferred_element_type=jnp.float32)
    o_ref[...] = jnp.where(y >= 0, y, y * _NEG)

  return pl.pallas_call(
      body,
      grid=(_N // _BLK,),
      in_specs=[
          pl.BlockSpec((2, _BLK, wi), lambda i: (0, i, 0)),
          pl.BlockSpec((wi, wo), lambda i: (0, 0)),
      ],
      out_specs=pl.BlockSpec((_BLK, wo), lambda i: (i, 0)),
      out_shape=jax.ShapeDtypeStruct((_N, wo), jnp.float32),
      name="mm_relu",
  )(p, w)


def _pool_softmax(p, w4p, batch2d):
  def body(p_ref, w_ref, b_ref, o_ref):
    s = p_ref[0] + p_ref[1]
    y = jnp.dot(s, w_ref[...], preferred_element_type=jnp.float32)
    h = jnp.where(y >= 0, y, y * _NEG)
    ccol = lax.broadcasted_iota(jnp.int32, (_N, 16), 1)
    h = jnp.where(ccol == 10, 1.0, h)
    gids = lax.broadcasted_iota(jnp.int32, (_NUM_GRAPHS, _N), 0)
    onehot = (gids == b_ref[...]).astype(jnp.float32)
    pooled = jnp.dot(onehot, h, preferred_element_type=jnp.float32,
                     precision=lax.Precision.HIGHEST)
    counts = jnp.maximum(pooled[:, 10:11], 1.0)
    means = pooled / counts
    gcol = lax.broadcasted_iota(jnp.int32, (_NUM_GRAPHS, 16), 1)
    valid = gcol < 10
    z = jnp.where(valid, means, -1e30)
    z = z - jnp.max(z, axis=1, keepdims=True)
    ez = jnp.where(valid, jnp.exp(z), 0.0)
    o_ref[...] = ez / jnp.sum(ez, axis=1, keepdims=True)

  return pl.pallas_call(
      body,
      in_specs=[
          pl.BlockSpec((2, _N, 32), lambda: (0, 0, 0)),
          pl.BlockSpec((32, 16), lambda: (0, 0)),
          pl.BlockSpec((1, _N), lambda: (0, 0)),
      ],
      out_specs=pl.BlockSpec((_NUM_GRAPHS, 16), lambda: (0, 0)),
      out_shape=jax.ShapeDtypeStruct((_NUM_GRAPHS, 16), jnp.float32),
      name="pool_softmax",
  )(p, w4p, batch2d)


def kernel(X1, L1_indices, L1_values, batch, W1, W2, W3, W4):
  row = L1_indices[0]
  col = L1_indices[1]
  npad = _PAD_NNZ - _NNZ
  pad_idx = (jnp.arange(npad, dtype=jnp.int32) * 131) % _N
  colf = jnp.concatenate([col, pad_idx])
  rowf = jnp.concatenate([row, pad_idx])
  valf = jnp.concatenate([L1_values, jnp.zeros((npad,), jnp.float32)])
  col128 = colf.reshape(_NT, _NCH128, _CH128)
  row128 = rowf.reshape(_NT, _NCH128, _CH128)
  val128 = valf.reshape(_NT, _NCH128, _CH128)
  col32 = colf.reshape(_NT, _NCH32, _CH32)
  row32 = rowf.reshape(_NT, _NCH32, _CH32)
  val32 = valf.reshape(_NT, _NCH32, _CH32)
  zeros128 = jnp.zeros((_N, 128), jnp.float32)
  zeros32 = jnp.zeros((_N, 32), jnp.float32)
  w4p = jnp.zeros((32, 16), jnp.float32).at[:, :10].set(W4)

  p = _spmm128(X1, col128, row128, val128, zeros128)
  h = _mm_relu(p, W1, 128, 32)
  p = _spmm32(h, col32, row32, val32, zeros32)
  h = _mm_relu(p, W2, 32, 32)
  p = _spmm32(h, col32, row32, val32, zeros32)
  h = _mm_relu(p, W3, 32, 32)
  p = _spmm32(h, col32, row32, val32, zeros32)
  out = _pool_softmax(p, w4p, batch.reshape(1, _N))
  return out[:, :10]
